# Initial kernel scaffold; baseline (speedup 1.0000x reference)
import jax
import jax.numpy as jnp
from jax.experimental import pallas as pl


def kernel(inputs):
    raise NotImplementedError("write your pallas kernel here")


# speedup vs baseline: 2.2327x; 2.2327x over previous
import functools

import jax
import jax.numpy as jnp
import numpy as np
from jax import lax
from jax.experimental import pallas as pl
from jax.experimental.pallas import tpu as pltpu
from jax.experimental.pallas import tpu_sc as plsc


def _quaternary_digits(n):
    d = 1
    while n >= 4:
        n //= 4
        d += 1
    return d


def _flat_shuffle_indices(batch, length):
    digits = _quaternary_digits(length - 1)
    i = np.arange(length, dtype=np.int64)
    mask = 4**digits - 1
    idx = ((i * 4) | (i >> (2 * (digits - 1)))) & mask
    b = np.arange(batch, dtype=np.int64)[:, None]
    flat = (b * length + idx[None, :]).reshape(-1)
    return np.asarray(flat, dtype=np.int32)


@functools.lru_cache(maxsize=None)
def _build(batch, length, dim):
    rows = batch * length
    info = plsc.get_sparse_core_info()
    nw = info.num_cores * info.num_subcores
    rows_per_w = rows // nw
    chunk = 64
    while rows_per_w % chunk:
        chunk //= 2
    nchunk = rows_per_w // chunk

    mesh = plsc.VectorSubcoreMesh(core_axis_name="c", subcore_axis_name="s")

    @functools.partial(
        pl.kernel,
        out_type=jax.ShapeDtypeStruct((rows, dim), jnp.float32),
        mesh=mesh,
        scratch_types=[
            pltpu.VMEM((chunk,), jnp.int32),
            pltpu.VMEM((chunk, dim), jnp.float32),
            pltpu.SemaphoreType.DMA,
        ],
    )
    def shuffle(x_hbm, idx_hbm, out_hbm, idx_v, rows_v, sem):
        wid = lax.axis_index("s") * info.num_cores + lax.axis_index("c")
        base = wid * rows_per_w

        def body(g, carry):
            off = base + g * chunk
            pltpu.sync_copy(idx_hbm.at[pl.ds(off, chunk)], idx_v)
            pltpu.async_copy(x_hbm.at[idx_v], rows_v, sem).wait()
            pltpu.sync_copy(rows_v, out_hbm.at[pl.ds(off, chunk)])
            return carry

        lax.fori_loop(0, nchunk, body, 0)

    return shuffle


def kernel(inputs):
    batch, length, dim = inputs.shape
    idx = jnp.asarray(_flat_shuffle_indices(batch, length))
    shuffle = _build(batch, length, dim)
    out = shuffle(inputs.reshape(batch * length, dim), idx)
    return out.reshape(batch, length, dim)


# speedup vs baseline: 2.3459x; 1.0507x over previous
import functools

import jax
import jax.numpy as jnp
import numpy as np
from jax import lax
from jax.experimental import pallas as pl
from jax.experimental.pallas import tpu as pltpu
from jax.experimental.pallas import tpu_sc as plsc


def _quaternary_digits(n):
    d = 1
    while n >= 4:
        n //= 4
        d += 1
    return d


def _flat_shuffle_indices(batch, length):
    digits = _quaternary_digits(length - 1)
    i = np.arange(length, dtype=np.int64)
    mask = 4**digits - 1
    idx = ((i * 4) | (i >> (2 * (digits - 1)))) & mask
    b = np.arange(batch, dtype=np.int64)[:, None]
    flat = (b * length + idx[None, :]).reshape(-1)
    return np.asarray(flat, dtype=np.int32)


def _chunk_rows(rows_per_w):
    chunk = 32
    while rows_per_w % chunk:
        chunk //= 2
    return chunk


@functools.lru_cache(maxsize=None)
def _build(batch, length, dim):
    rows = batch * length
    info = plsc.get_sparse_core_info()
    nw = info.num_cores * info.num_subcores
    rows_per_w = rows // nw
    chunk = _chunk_rows(rows_per_w)
    nchunk = rows_per_w // chunk

    mesh = plsc.VectorSubcoreMesh(core_axis_name="c", subcore_axis_name="s")

    @functools.partial(
        pl.kernel,
        out_type=jax.ShapeDtypeStruct((rows, dim), jnp.float32),
        mesh=mesh,
        scratch_types=[
            pltpu.VMEM((nchunk, chunk), jnp.int32),
            pltpu.VMEM((chunk, dim), jnp.float32),
            pltpu.VMEM((chunk, dim), jnp.float32),
            pltpu.SemaphoreType.DMA,
            pltpu.SemaphoreType.DMA,
        ],
    )
    def shuffle(x_hbm, idx_hbm, out_hbm, idx_v, rows0, rows1, sem0, sem1):
        wid = lax.axis_index("s") * info.num_cores + lax.axis_index("c")
        base = wid * rows_per_w
        bufs = (rows0, rows1)
        sems = (sem0, sem1)

        pltpu.sync_copy(idx_hbm.at[wid], idx_v)
        pending = [None, None]
        pending[0] = pltpu.async_copy(x_hbm.at[idx_v.at[0]], bufs[0], sems[0])
        for g in range(nchunk):
            p = g % 2
            if g + 1 < nchunk:
                q = (g + 1) % 2
                pending[q] = pltpu.async_copy(
                    x_hbm.at[idx_v.at[g + 1]], bufs[q], sems[q]
                )
            pending[p].wait()
            pltpu.sync_copy(bufs[p], out_hbm.at[pl.ds(base + g * chunk, chunk)])

    return shuffle


def kernel(inputs):
    batch, length, dim = inputs.shape
    rows = batch * length
    shuffle = _build(batch, length, dim)
    info = plsc.get_sparse_core_info()
    nw = info.num_cores * info.num_subcores
    chunk = _chunk_rows(rows // nw)
    idx = jnp.asarray(_flat_shuffle_indices(batch, length)).reshape(nw, -1, chunk)
    out = shuffle(inputs.reshape(rows, dim), idx)
    return out.reshape(batch, length, dim)
